# Initial kernel scaffold; baseline (speedup 1.0000x reference)
import functools
import jax
import jax.numpy as jnp
from jax import lax
from jax.experimental import pallas as pl
from jax.experimental.pallas import tpu as pltpu
from jax.experimental.pallas import tpu_sc as plsc

_B = 16384
_L = 50
_D = 16
_NC = 2
_NS = 16
_NW = _NC * _NS
_RPW = _B // _NW
_CH = 32
_NCHUNK = _RPW // _CH

_SCAL_F = [0, 2, 4, 6, 8, 10, 12, 14, 15, 16, 18]
_POOL_F = [1, 3, 5, 7, 9, 11, 13, 17, 19]


def _sc_body(*refs):
    sidx = refs[0:11]
    lidx = refs[11:20]
    dtab = refs[20:33]
    wtab = refs[33:40]
    x2, ws, wlv = refs[40:43]
    idx1, srows, idxL, rows, pooled, wvals, wlrows, sem = refs[43:51]

    wid = lax.axis_index("s") * _NC + lax.axis_index("c")
    base = wid * _RPW
    base50 = base * _L

    scal_tabs = [0, 1, 2, 3, 4, 5, 6, 7, 8, 9, 10]
    for k in range(11):
        pltpu.sync_copy(sidx[k].at[pl.ds(base, _RPW)], idx1)
        pltpu.async_copy(dtab[scal_tabs[k]].at[idx1], srows, sem).wait()
        pltpu.sync_copy(srows, x2.at[pl.ds(_SCAL_F[k] * _B + base, _RPW), :])

    pool_tabs = [0, 1, 2, 3, 4, 5, 6, 11, 12]
    for k in range(9):
        tab = dtab[pool_tabs[k]]
        lk = lidx[k]

        def chunk_body(c, _, tab=tab, lk=lk):
            off = base50 + c * (_CH * _L)
            pltpu.sync_copy(lk.at[pl.ds(off, _CH * _L)], idxL)
            pltpu.async_copy(tab.at[idxL], rows, sem).wait()

            def sbody(s, _):
                rb = s * _L
                acc = rows[rb]
                for j in range(1, _L):
                    acc = acc + rows[rb + j]
                pooled[c * _CH + s] = acc
                return 0

            lax.fori_loop(0, _CH, sbody, 0)
            return 0

        lax.fori_loop(0, _NCHUNK, chunk_body, 0)
        pltpu.sync_copy(pooled, x2.at[pl.ds(_POOL_F[k] * _B + base, _RPW), :])

    wide_map = [(0, 7), (1, 0), (2, 1), (3, 2), (4, 4), (5, 10)]
    for w, (ti, si) in enumerate(wide_map):
        pltpu.sync_copy(sidx[si].at[pl.ds(base, _RPW)], idx1)
        pltpu.async_copy(wtab[ti].at[idx1], wvals, sem).wait()
        pltpu.sync_copy(wvals, ws.at[pl.ds(w * _B + base, _RPW), :])

    def wl_body(c, _):
        off = base50 + c * (_CH * _L)
        pltpu.sync_copy(lidx[0].at[pl.ds(off, _CH * _L)], idxL)
        pltpu.async_copy(wtab[6].at[idxL], wlrows, sem).wait()
        pltpu.sync_copy(wlrows, wlv.at[pl.ds(off, _CH * _L), :])
        return 0

    lax.fori_loop(0, _NCHUNK, wl_body, 0)


@jax.jit
def _sc_gather(sidx_list, lidx_list, dtab_list, wtab_list):
    mesh = plsc.VectorSubcoreMesh(core_axis_name="c", subcore_axis_name="s",
                                  num_cores=_NC, num_subcores=_NS)
    f = pl.kernel(
        _sc_body,
        out_type=(
            jax.ShapeDtypeStruct((20 * _B, _D), jnp.float32),
            jax.ShapeDtypeStruct((6 * _B, 1), jnp.float32),
            jax.ShapeDtypeStruct((_B * _L, 1), jnp.float32),
        ),
        mesh=mesh,
        scratch_types=(
            pltpu.VMEM((_RPW,), jnp.int32),
            pltpu.VMEM((_RPW, _D), jnp.float32),
            pltpu.VMEM((_CH * _L,), jnp.int32),
            pltpu.VMEM((_CH * _L, _D), jnp.float32),
            pltpu.VMEM((_RPW, _D), jnp.float32),
            pltpu.VMEM((_RPW, 1), jnp.float32),
            pltpu.VMEM((_CH * _L, 1), jnp.float32),
            pltpu.SemaphoreType.DMA,
        ),
    )
    return f(*sidx_list, *lidx_list, *dtab_list, *wtab_list)


_BBLK = 256


def _tc_body(x3, sids, l0, l1, l2, l3, l4, l5, l6, l7, l8, cont, t0, wsv,
             wlv, wt0, w1t, b1, w2t, b2, w3t, b3, w4t, b4, w5, b5, out):
    lists = [l0, l1, l2, l3, l4, l5, l6, l7, l8]
    zl = [jnp.sum((lr[...] == 0).astype(jnp.float32), axis=1) for lr in lists]

    pieces = []
    li = 0
    si = 0
    for f in range(20):
        raw = x3[f]
        if f in (1, 3, 5, 7, 9, 11, 13, 17, 19):
            z = zl[li]
            li += 1
        else:
            z = (sids[si] == 0).astype(jnp.float32)
            si += 1
        pieces.append(raw - z[:, None] * t0[f][None, :])
    pieces.append(cont[...])
    x = jnp.concatenate(pieces, axis=1)
    nrm = jnp.sqrt(jnp.sum(x * x, axis=1, keepdims=True))
    x = x / jnp.maximum(nrm, 1e-12)

    h = jnp.maximum(jnp.dot(x, w1t[...], preferred_element_type=jnp.float32)
                    + b1[...], 0.0)
    h = jnp.maximum(jnp.dot(h, w2t[...], preferred_element_type=jnp.float32)
                    + b2[...], 0.0)
    h = jnp.maximum(jnp.dot(h, w3t[...], preferred_element_type=jnp.float32)
                    + b3[...], 0.0)
    h = jnp.maximum(jnp.dot(h, w4t[...], preferred_element_type=jnp.float32)
                    + b4[...], 0.0)
    deep = jnp.sum(h * w5[0][None, :], axis=1) + b5[0, 0]

    wide_si = [7, 0, 1, 2, 4, 10]
    wide = jnp.zeros_like(deep)
    for w in range(6):
        z = (sids[wide_si[w]] == 0).astype(jnp.float32)
        wide = wide + wsv[w] - z * wt0[0, w]
    wide = wide + jnp.sum(wlv[...], axis=1) - zl[0] * wt0[0, 6]
    wide = wide + cont[:, 0] + cont[:, 1] + cont[:, 2] + cont[:, 3]

    out[...] = jax.nn.sigmoid(deep + wide)[:, None]


@jax.jit
def _tc_mlp(x3, sids, lists, cont, t0, wsv, wlv, wt0, ws, bs):
    nblk = _B // _BBLK
    in_specs = [
        pl.BlockSpec((20, _BBLK, _D), lambda i: (0, i, 0)),
        pl.BlockSpec((11, _BBLK), lambda i: (0, i)),
    ]
    in_specs += [pl.BlockSpec((_BBLK, _L), lambda i: (i, 0))] * 9
    in_specs += [
        pl.BlockSpec((_BBLK, 6), lambda i: (i, 0)),
        pl.BlockSpec((20, _D), lambda i: (0, 0)),
        pl.BlockSpec((6, _BBLK), lambda i: (0, i)),
        pl.BlockSpec((_BBLK, _L), lambda i: (i, 0)),
        pl.BlockSpec((1, 8), lambda i: (0, 0)),
    ]
    wb = []
    for wt, bt in zip(ws, bs):
        in_specs.append(pl.BlockSpec(wt.shape, lambda i: (0, 0)))
        in_specs.append(pl.BlockSpec(bt.shape, lambda i: (0, 0)))
        wb += [wt, bt]
    return pl.pallas_call(
        _tc_body,
        grid=(nblk,),
        in_specs=in_specs,
        out_specs=pl.BlockSpec((_BBLK, 1), lambda i: (i, 0)),
        out_shape=jax.ShapeDtypeStruct((_B, 1), jnp.float32),
    )(x3, sids, *lists, cont, t0, wsv, wlv, wt0, *wb)


def kernel(user_id, district_id, times, shop_id, item_id, category_1_id,
           merge_standard_food_id, brand_id, shop_aoi_id, shop_geohash_6,
           timediff_list, shop_id_list, item_id_list, category_1_id_list,
           merge_standard_food_id_list, brand_id_list, shop_aoi_id_list,
           shop_geohash6_list, time_type, time_type_list, rank_7, rank_30,
           rank_90, hours, price_list, hours_list, emb_shop, emb_item,
           emb_cat1, emb_food, emb_brand, emb_aoi, emb_geo, emb_user,
           emb_district, emb_times, emb_timediff, emb_time_type,
           emb_time_type_list, lin_user, lin_shop, lin_item, lin_cat1,
           lin_brand, lin_time_type, lin_shop_list, W1, b1, W2, b2, W3, b3,
           W4, b4, W5, b5):
    sidx = [shop_id, item_id, category_1_id, merge_standard_food_id,
            brand_id, shop_aoi_id, shop_geohash_6, user_id, district_id,
            times, time_type]
    lists = [shop_id_list, item_id_list, category_1_id_list,
             merge_standard_food_id_list, brand_id_list, shop_aoi_id_list,
             shop_geohash6_list, timediff_list, time_type_list]
    lidx = [l.reshape(-1) for l in lists]
    dtab = [emb_shop, emb_item, emb_cat1, emb_food, emb_brand, emb_aoi,
            emb_geo, emb_user, emb_district, emb_times, emb_time_type,
            emb_timediff, emb_time_type_list]
    wtab = [lin_user, lin_shop, lin_item, lin_cat1, lin_brand,
            lin_time_type, lin_shop_list]

    x2, wsf, wlvf = _sc_gather(sidx, lidx, dtab, wtab)

    x3 = x2.reshape(20, _B, _D)
    wsv = wsf.reshape(6, _B)
    wlv = wlvf.reshape(_B, _L)
    sids = jnp.stack(sidx)
    cont = jnp.concatenate([rank_7, rank_30, rank_90, hours, price_list,
                            hours_list], axis=1)
    t0 = jnp.stack([
        emb_shop[0], emb_shop[0], emb_item[0], emb_item[0], emb_cat1[0],
        emb_cat1[0], emb_food[0], emb_food[0], emb_brand[0], emb_brand[0],
        emb_aoi[0], emb_aoi[0], emb_geo[0], emb_geo[0], emb_user[0],
        emb_district[0], emb_times[0], emb_timediff[0], emb_time_type[0],
        emb_time_type_list[0]])
    wt0 = jnp.stack([lin_user[0, 0], lin_shop[0, 0], lin_item[0, 0],
                     lin_cat1[0, 0], lin_brand[0, 0], lin_time_type[0, 0],
                     lin_shop_list[0, 0],
                     jnp.float32(0.0)]).reshape(1, 8)
    ws = [W1.T, W2.T, W3.T, W4.T, W5]
    bs = [b1.reshape(1, -1), b2.reshape(1, -1), b3.reshape(1, -1),
          b4.reshape(1, -1), b5.reshape(1, 1)]
    out = _tc_mlp(x3, sids, lists, cont, t0, wsv, wlv, wt0, ws, bs)
    return out.reshape(-1)


# speedup vs baseline: 5.4386x; 5.4386x over previous
import functools
import jax
import jax.numpy as jnp
from jax import lax
from jax.experimental import pallas as pl
from jax.experimental.pallas import tpu as pltpu
from jax.experimental.pallas import tpu_sc as plsc

_B = 16384
_L = 50
_D = 16
_NC = 2
_NS = 16
_NW = _NC * _NS
_RPW = _B // _NW
_CH = 32
_NCHUNK = _RPW // _CH

_SCAL_F = [0, 2, 4, 6, 8, 10, 12, 14, 15, 16, 18]
_POOL_F = [1, 3, 5, 7, 9, 11, 13, 17, 19]


def _sc_body(*refs):
    sidx = refs[0:11]
    lidx = refs[11:20]
    dtab = refs[20:33]
    wtab = refs[33:40]
    x2, ws, wlv = refs[40:43]
    idx1, srows, idxL, rows, pooled, wvals, wlrows, sem = refs[43:51]

    wid = lax.axis_index("s") * _NC + lax.axis_index("c")
    base = wid * _RPW
    base50 = base * _L

    scal_tabs = [0, 1, 2, 3, 4, 5, 6, 7, 8, 9, 10]
    for k in range(11):
        pltpu.sync_copy(sidx[k].at[pl.ds(base, _RPW)], idx1)
        pltpu.async_copy(dtab[scal_tabs[k]].at[idx1], srows, sem).wait()
        pltpu.sync_copy(srows, x2.at[pl.ds(_SCAL_F[k] * _B + base, _RPW), :])

    pool_tabs = [0, 1, 2, 3, 4, 5, 6, 11, 12]
    for k in range(9):
        tab = dtab[pool_tabs[k]]
        lk = lidx[k]

        def chunk_body(c, _, tab=tab, lk=lk):
            off = base50 + c * (_CH * _L)
            pltpu.sync_copy(lk.at[pl.ds(off, _CH * _L)], idxL)
            pltpu.async_copy(tab.at[idxL], rows, sem).wait()

            def sbody(s, _):
                rb = s * _L
                acc = rows[rb]
                for j in range(1, _L):
                    acc = acc + rows[rb + j]
                pooled[c * _CH + s] = acc
                return 0

            lax.fori_loop(0, _CH, sbody, 0)
            return 0

        lax.fori_loop(0, _NCHUNK, chunk_body, 0)
        pltpu.sync_copy(pooled, x2.at[pl.ds(_POOL_F[k] * _B + base, _RPW), :])

    wide_map = [(0, 7), (1, 0), (2, 1), (3, 2), (4, 4), (5, 10)]
    for w, (ti, si) in enumerate(wide_map):
        pltpu.sync_copy(sidx[si].at[pl.ds(base, _RPW)], idx1)
        pltpu.async_copy(wtab[ti].at[idx1], wvals, sem).wait()
        pltpu.sync_copy(wvals, ws.at[pl.ds(w * _B + base, _RPW), :])

    def wl_body(c, _):
        off = base50 + c * (_CH * _L)
        pltpu.sync_copy(lidx[0].at[pl.ds(off, _CH * _L)], idxL)
        pltpu.async_copy(wtab[6].at[idxL], wlrows, sem).wait()
        pltpu.sync_copy(wlrows, wlv.at[pl.ds(off, _CH * _L), :])
        return 0

    lax.fori_loop(0, _NCHUNK, wl_body, 0)


@jax.jit
def _sc_gather(sidx_list, lidx_list, dtab_list, wtab_list):
    mesh = plsc.VectorSubcoreMesh(core_axis_name="c", subcore_axis_name="s",
                                  num_cores=_NC, num_subcores=_NS)
    f = pl.kernel(
        _sc_body,
        out_type=(
            jax.ShapeDtypeStruct((20 * _B, _D), jnp.float32),
            jax.ShapeDtypeStruct((6 * _B, 1), jnp.float32),
            jax.ShapeDtypeStruct((_B * _L, 1), jnp.float32),
        ),
        mesh=mesh,
        scratch_types=(
            pltpu.VMEM((_RPW,), jnp.int32),
            pltpu.VMEM((_RPW, _D), jnp.float32),
            pltpu.VMEM((_CH * _L,), jnp.int32),
            pltpu.VMEM((_CH * _L, _D), jnp.float32),
            pltpu.VMEM((_RPW, _D), jnp.float32),
            pltpu.VMEM((_RPW, 1), jnp.float32),
            pltpu.VMEM((_CH * _L, 1), jnp.float32),
            pltpu.SemaphoreType.DMA,
        ),
        compiler_params=pltpu.CompilerParams(use_tc_tiling_on_sc=False),
    )
    return f(*sidx_list, *lidx_list, *dtab_list, *wtab_list)


_BBLK = 256


def _tc_body(x3, sids, l0, l1, l2, l3, l4, l5, l6, l7, l8, cont, t0, wsv,
             wlv, wt0, w1t, b1, w2t, b2, w3t, b3, w4t, b4, w5, b5, out):
    lists = [l0, l1, l2, l3, l4, l5, l6, l7, l8]
    zl = [jnp.sum((lr[...] == 0).astype(jnp.float32), axis=1) for lr in lists]

    pieces = []
    li = 0
    si = 0
    for f in range(20):
        raw = x3[f]
        if f in (1, 3, 5, 7, 9, 11, 13, 17, 19):
            z = zl[li]
            li += 1
        else:
            z = (sids[si] == 0).astype(jnp.float32)
            si += 1
        pieces.append(raw - z[:, None] * t0[f][None, :])
    pieces.append(cont[...])
    x = jnp.concatenate(pieces, axis=1)
    nrm = jnp.sqrt(jnp.sum(x * x, axis=1, keepdims=True))
    x = x / jnp.maximum(nrm, 1e-12)

    h = jnp.maximum(jnp.dot(x, w1t[...], preferred_element_type=jnp.float32)
                    + b1[...], 0.0)
    h = jnp.maximum(jnp.dot(h, w2t[...], preferred_element_type=jnp.float32)
                    + b2[...], 0.0)
    h = jnp.maximum(jnp.dot(h, w3t[...], preferred_element_type=jnp.float32)
                    + b3[...], 0.0)
    h = jnp.maximum(jnp.dot(h, w4t[...], preferred_element_type=jnp.float32)
                    + b4[...], 0.0)
    deep = jnp.sum(h * w5[0][None, :], axis=1) + b5[0, 0]

    wide_si = [7, 0, 1, 2, 4, 10]
    wide = jnp.zeros_like(deep)
    for w in range(6):
        z = (sids[wide_si[w]] == 0).astype(jnp.float32)
        wide = wide + wsv[w] - z * wt0[0, w]
    wide = wide + jnp.sum(wlv[...], axis=1) - zl[0] * wt0[0, 6]
    wide = wide + cont[:, 0] + cont[:, 1] + cont[:, 2] + cont[:, 3]

    out[...] = jax.nn.sigmoid(deep + wide)[:, None]


@jax.jit
def _tc_mlp(x3, sids, lists, cont, t0, wsv, wlv, wt0, ws, bs):
    nblk = _B // _BBLK
    in_specs = [
        pl.BlockSpec((20, _BBLK, _D), lambda i: (0, i, 0)),
        pl.BlockSpec((11, _BBLK), lambda i: (0, i)),
    ]
    in_specs += [pl.BlockSpec((_BBLK, _L), lambda i: (i, 0))] * 9
    in_specs += [
        pl.BlockSpec((_BBLK, 6), lambda i: (i, 0)),
        pl.BlockSpec((20, _D), lambda i: (0, 0)),
        pl.BlockSpec((6, _BBLK), lambda i: (0, i)),
        pl.BlockSpec((_BBLK, _L), lambda i: (i, 0)),
        pl.BlockSpec((1, 8), lambda i: (0, 0)),
    ]
    wb = []
    for wt, bt in zip(ws, bs):
        in_specs.append(pl.BlockSpec(wt.shape, lambda i: (0, 0)))
        in_specs.append(pl.BlockSpec(bt.shape, lambda i: (0, 0)))
        wb += [wt, bt]
    return pl.pallas_call(
        _tc_body,
        grid=(nblk,),
        in_specs=in_specs,
        out_specs=pl.BlockSpec((_BBLK, 1), lambda i: (i, 0)),
        out_shape=jax.ShapeDtypeStruct((_B, 1), jnp.float32),
    )(x3, sids, *lists, cont, t0, wsv, wlv, wt0, *wb)


def kernel(user_id, district_id, times, shop_id, item_id, category_1_id,
           merge_standard_food_id, brand_id, shop_aoi_id, shop_geohash_6,
           timediff_list, shop_id_list, item_id_list, category_1_id_list,
           merge_standard_food_id_list, brand_id_list, shop_aoi_id_list,
           shop_geohash6_list, time_type, time_type_list, rank_7, rank_30,
           rank_90, hours, price_list, hours_list, emb_shop, emb_item,
           emb_cat1, emb_food, emb_brand, emb_aoi, emb_geo, emb_user,
           emb_district, emb_times, emb_timediff, emb_time_type,
           emb_time_type_list, lin_user, lin_shop, lin_item, lin_cat1,
           lin_brand, lin_time_type, lin_shop_list, W1, b1, W2, b2, W3, b3,
           W4, b4, W5, b5):
    sidx = [shop_id, item_id, category_1_id, merge_standard_food_id,
            brand_id, shop_aoi_id, shop_geohash_6, user_id, district_id,
            times, time_type]
    lists = [shop_id_list, item_id_list, category_1_id_list,
             merge_standard_food_id_list, brand_id_list, shop_aoi_id_list,
             shop_geohash6_list, timediff_list, time_type_list]
    lidx = [l.reshape(-1) for l in lists]
    dtab = [emb_shop, emb_item, emb_cat1, emb_food, emb_brand, emb_aoi,
            emb_geo, emb_user, emb_district, emb_times, emb_time_type,
            emb_timediff, emb_time_type_list]
    wtab = [lin_user, lin_shop, lin_item, lin_cat1, lin_brand,
            lin_time_type, lin_shop_list]

    x2, wsf, wlvf = _sc_gather(sidx, lidx, dtab, wtab)

    x3 = x2.reshape(20, _B, _D)
    wsv = wsf.reshape(6, _B)
    wlv = wlvf.reshape(_B, _L)
    sids = jnp.stack(sidx)
    cont = jnp.concatenate([rank_7, rank_30, rank_90, hours, price_list,
                            hours_list], axis=1)
    t0 = jnp.stack([
        emb_shop[0], emb_shop[0], emb_item[0], emb_item[0], emb_cat1[0],
        emb_cat1[0], emb_food[0], emb_food[0], emb_brand[0], emb_brand[0],
        emb_aoi[0], emb_aoi[0], emb_geo[0], emb_geo[0], emb_user[0],
        emb_district[0], emb_times[0], emb_timediff[0], emb_time_type[0],
        emb_time_type_list[0]])
    wt0 = jnp.stack([lin_user[0, 0], lin_shop[0, 0], lin_item[0, 0],
                     lin_cat1[0, 0], lin_brand[0, 0], lin_time_type[0, 0],
                     lin_shop_list[0, 0],
                     jnp.float32(0.0)]).reshape(1, 8)
    ws = [W1.T, W2.T, W3.T, W4.T, W5]
    bs = [b1.reshape(1, -1), b2.reshape(1, -1), b3.reshape(1, -1),
          b4.reshape(1, -1), b5.reshape(1, 1)]
    out = _tc_mlp(x3, sids, lists, cont, t0, wsv, wlv, wt0, ws, bs)
    return out.reshape(-1)
